# baseline (speedup 1.0000x reference)
import functools
import jax
import jax.numpy as jnp
from jax import lax
from jax.experimental import pallas as pl
from jax.experimental.pallas import tpu as pltpu, tpu_sc as plsc

N = 2097152
B = 2048
NC = 2
NS = 16
T = NC * NS
CHUNK = N // T
W = 8192
NW = CHUNK // W
VPW = W // 16
SCAN_BINS = B // NS
H = N // 2
FLUSH = H // NS

_mesh = plsc.VectorSubcoreMesh(core_axis_name="c", subcore_axis_name="s")
_cp = pltpu.CompilerParams(needs_layout_passes=False, use_tc_tiling_on_sc=True)


def _iota16():
  return lax.iota(jnp.int32, 16)


def _col16(v):
  return jnp.full((16,), v, jnp.int32)


def _fill(ref, n, value):
  v = jnp.full((16,), value, jnp.int32)
  for i in range(n // 16):
    ref[pl.ds(i * 16, 16)] = v


def _build_idx(ref, n, scale, offset):
  ii = _iota16()
  for i in range(n // 16):
    ref[pl.ds(i * 16, 16)] = (ii + i * 16) * scale + offset


def _hist_b_body(bcol, ht, bufv, histv):
  w = lax.axis_index("c") * NS + lax.axis_index("s")
  _fill(histv, B, 0)

  @pl.loop(0, NW)
  def _win(win):
    pltpu.sync_copy(bcol.at[pl.ds(w * CHUNK + win * W, W)], bufv)
    for j in range(VPW):
      b16 = bufv[pl.ds(j * 16, 16)]
      cnt, lastm = plsc.scan_count(b16)
      plsc.addupdate_scatter(histv, [b16], cnt, mask=lastm)

  pltpu.sync_copy(histv, ht.at[pl.ds(w * B, B)])


_hist_b = pl.kernel(
    _hist_b_body,
    out_type=jax.ShapeDtypeStruct((T * B,), jnp.int32),
    mesh=_mesh,
    compiler_params=_cp,
    scratch_types=[
        pltpu.VMEM((W,), jnp.int32),
        pltpu.VMEM((B,), jnp.int32),
    ],
)


def _scan_body(two_inputs, *refs):
  if two_inputs:
    h0, h1, s_out, h0v, h1v, sv, totv, shv, sh = refs
  else:
    h0, s_out, h0v, sv, totv, shv, sh = refs
  c = lax.axis_index("c")
  s = lax.axis_index("s")
  ii = _iota16()

  @pl.when(c == 0)
  def _():
    for t in range(T):
      src = pl.ds(t * B + s * SCAN_BINS, SCAN_BINS)
      dst = pl.ds(t * SCAN_BINS, SCAN_BINS)
      pltpu.sync_copy(h0.at[src], h0v.at[dst])
      if two_inputs:
        pltpu.sync_copy(h1.at[src], h1v.at[dst])
    if two_inputs:
      for i in range(T * SCAN_BINS // 16):
        sl = pl.ds(i * 16, 16)
        h0v[sl] = h0v[sl] + h1v[sl]

    @pl.loop(0, SCAN_BINS, init_carry=jnp.int32(0))
    def _bin(bin_local, carry):
      lo = plsc.load_gather(h0v, [ii * SCAN_BINS + bin_local])
      hi = plsc.load_gather(h0v, [(ii + 16) * SCAN_BINS + bin_local])
      cs_lo = plsc.cumsum(lo)
      cs_hi = plsc.cumsum(hi)
      tot_lo = jnp.sum(lo)
      off = pl.multiple_of(bin_local * T, T)
      sv[pl.ds(off, 16)] = cs_lo - lo + carry
      sv[pl.ds(off + 16, 16)] = cs_hi - hi + (carry + tot_lo)
      return carry + tot_lo + jnp.sum(hi)

    total = _bin
    totv[pl.ds(0, 16)] = jnp.where(ii == 0, total, 0)
    pltpu.sync_copy(totv.at[pl.ds(0, 8)], sh.at[pl.ds(s * 8, 8)])

  plsc.subcore_barrier()

  @pl.when(c == 0)
  def _():
    pltpu.sync_copy(sh, shv)
    totals = plsc.load_gather(shv, [ii * 8])
    base = jnp.sum(jnp.where(ii < s, totals, 0))
    for i in range(T * SCAN_BINS // 16):
      sl = pl.ds(i * 16, 16)
      sv[sl] = sv[sl] + base
    pltpu.sync_copy(sv, s_out.at[pl.ds(s * T * SCAN_BINS, T * SCAN_BINS)])


def _make_scan(two_inputs):
  scratch = [
      pltpu.VMEM((T * SCAN_BINS,), jnp.int32),
      pltpu.VMEM((T * SCAN_BINS,), jnp.int32),
      pltpu.VMEM((16,), jnp.int32),
      pltpu.VMEM((NS * 8,), jnp.int32),
      pltpu.VMEM_SHARED((NS * 8,), jnp.int32),
  ]
  if two_inputs:
    scratch.insert(1, pltpu.VMEM((T * SCAN_BINS,), jnp.int32))
  return pl.kernel(
      functools.partial(_scan_body, two_inputs),
      out_type=jax.ShapeDtypeStruct((T * B,), jnp.int32),
      mesh=_mesh,
      compiler_params=_cp,
      scratch_types=scratch,
  )


_scan1 = _make_scan(False)
_scan2 = _make_scan(True)


DUMP = H
HIST0 = H + 256
SP_WORDS = HIST0 + T * B


def _pass_body(pass_id, *refs):
  if pass_id == 1:
    (bcol, rcol, s_in, zeros, out_a, out_b, ht_next,
     bcv, rcv, splitsv, offv, gidxv, abuf, bbuf, s1buf, s2buf, hbuf, onesbuf,
     spd, sem) = refs
  elif pass_id == 2:
    (in_a, in_b, s_in, zeros, out_a, out_b, ht_next,
     av, bv, offv, gidxv, abuf, bbuf, s1buf, s2buf, hbuf, onesbuf,
     spd, sem) = refs
  else:
    (in_a, in_b, s_in, pos_out,
     av, bv, offv, gidxv, abuf, s1buf,
     spd, sem) = refs
  c = lax.axis_index("c")
  s = lax.axis_index("s")
  ii = _iota16()

  if pass_id != 3:
    @pl.when(s == 0)
    def _():
      pltpu.sync_copy(zeros, spd.at[pl.ds(HIST0, T * B)])
    if pass_id == 1:
      _build_idx(gidxv, B, T, 0)
      pltpu.async_copy(s_in.at[gidxv], splitsv, sem).wait()
    _fill(onesbuf, W, 1)
  plsc.subcore_barrier()

  hbase = c * H
  n_rounds = 1 if pass_id == 3 else 2
  for rnd in range(n_rounds):
    for half in range(2):
      k = 2 * s + half
      _build_idx(gidxv, B, T, k)
      pltpu.async_copy(s_in.at[gidxv], offv, sem).wait()

      @pl.loop(0, NW)
      def _win(win):
        base = k * CHUNK + win * W
        if pass_id == 1:
          pltpu.sync_copy(bcol.at[pl.ds(base, W)], bcv)
          if rnd == 0:
            pltpu.sync_copy(rcol.at[pl.ds(base, W)], rcv)
        else:
          pltpu.sync_copy(in_a.at[pl.ds(base, W)], av)
          if pass_id == 3 or rnd == 1:
            pltpu.sync_copy(in_b.at[pl.ds(base, W)], bv)

        @pl.loop(0, VPW, unroll=4)
        def _vreg(j):
          sl = pl.ds(pl.multiple_of(j * 16, 16), 16)
          if pass_id == 1:
            b16 = bcv[sl]
            d16 = b16
            if rnd == 0:
              key2 = rcv[sl] + plsc.load_gather(splitsv, [b16])
            idxval = base + j * 16 + ii
          else:
            key2 = av[sl]
            idxval = bv[sl]
            d16 = (key2 & (B - 1)) if pass_id == 2 else (key2 >> 11)
          cnt, lastm = plsc.scan_count(d16)
          dest = plsc.load_gather(offv, [d16]) + (cnt - 1)
          plsc.addupdate_scatter(offv, [d16], cnt, mask=lastm)
          if pass_id == 3:
            loc = idxval - hbase
            owned = (loc >= 0) & (loc < H)
            s1buf[sl] = jnp.where(owned, loc, DUMP + ii)
            abuf[sl] = dest
          else:
            loc = dest - hbase
            owned = (loc >= 0) & (loc < H)
            s1buf[sl] = jnp.where(owned, loc, DUMP + ii)
            abuf[sl] = key2 if rnd == 0 else idxval
            if rnd == 0:
              nxt = (key2 & (B - 1)) if pass_id == 1 else (key2 >> 11)
              hbuf[sl] = HIST0 + ((dest >> 16) << 11) + nxt
        d1 = pltpu.async_copy(abuf, spd.at[s1buf], sem)
        if pass_id != 3 and rnd == 0:
          @pl.when(c == 0)
          def _():
            pltpu.sync_copy(onesbuf, spd.at[hbuf], add=True)
        d1.wait()

    plsc.subcore_barrier()
    if pass_id == 3:
      dst = pos_out
    else:
      dst = out_a if rnd == 0 else out_b
    pltpu.sync_copy(spd.at[pl.ds(s * FLUSH, FLUSH)],
                    dst.at[pl.ds(hbase + s * FLUSH, FLUSH)])
    plsc.subcore_barrier()

  if pass_id != 3:
    @pl.when(s == 0)
    def _():
      pltpu.sync_copy(spd.at[pl.ds(HIST0, T * B)],
                      ht_next.at[pl.ds(c * (T * B), T * B)])


def _make_pass(pass_id):
  plane = jax.ShapeDtypeStruct((N,), jnp.int32)
  if pass_id == 3:
    out_type = plane
    scratch = [
        pltpu.VMEM((W,), jnp.int32),
        pltpu.VMEM((W,), jnp.int32),
        pltpu.VMEM((B,), jnp.int32),
        pltpu.VMEM((B,), jnp.int32),
        pltpu.VMEM((W,), jnp.int32),
        pltpu.VMEM((W,), jnp.int32),
        pltpu.VMEM_SHARED((H + 256,), jnp.int32),
        pltpu.SemaphoreType.DMA,
    ]
  else:
    out_type = (plane, plane,
                jax.ShapeDtypeStruct((NC * T * B,), jnp.int32))
    first = ([pltpu.VMEM((W,), jnp.int32), pltpu.VMEM((W,), jnp.int32),
              pltpu.VMEM((B,), jnp.int32)]
             if pass_id == 1 else
             [pltpu.VMEM((W,), jnp.int32), pltpu.VMEM((W,), jnp.int32)])
    scratch = first + [
        pltpu.VMEM((B,), jnp.int32),
        pltpu.VMEM((B,), jnp.int32),
        pltpu.VMEM((W,), jnp.int32),
        pltpu.VMEM((W,), jnp.int32),
        pltpu.VMEM((W,), jnp.int32),
        pltpu.VMEM((W,), jnp.int32),
        pltpu.VMEM((W,), jnp.int32),
        pltpu.VMEM((W,), jnp.int32),
        pltpu.VMEM_SHARED((SP_WORDS,), jnp.int32),
        pltpu.SemaphoreType.DMA,
    ]
  return pl.kernel(
      functools.partial(_pass_body, pass_id),
      out_type=out_type,
      mesh=_mesh,
      compiler_params=_cp,
      scratch_types=scratch,
  )


_pass1 = _make_pass(1)
_pass2 = _make_pass(2)
_pass3 = _make_pass(3)


def _dist_a_body(bcol, vals, pos, rid, ew,
                 bufv, vv, posv, bbuf, vbuf, s1buf, s2buf, spd, sem):
  c = lax.axis_index("c")
  s = lax.axis_index("s")
  ii = _iota16()

  hbase = c * H
  for rnd in range(2):
    for half in range(2):
      k = 2 * s + half

      @pl.loop(0, NW)
      def _win(win):
        base = k * CHUNK + win * W
        src_ref = bcol if rnd == 0 else vals
        pltpu.sync_copy(src_ref.at[pl.ds(base, W)], vv)
        pltpu.sync_copy(pos.at[pl.ds(base, W)], posv)

        @pl.loop(0, VPW, unroll=4)
        def _vreg(j):
          sl = pl.ds(pl.multiple_of(j * 16, 16), 16)
          p = posv[sl]
          loc = p - hbase
          owned = (loc >= 0) & (loc < H)
          s1buf[sl] = jnp.where(owned, loc, DUMP + ii)
          vbuf[sl] = vv[sl]
        pltpu.async_copy(vbuf, spd.at[s1buf], sem).wait()

    plsc.subcore_barrier()
    dst = rid if rnd == 0 else ew
    pltpu.sync_copy(spd.at[pl.ds(s * FLUSH, FLUSH)],
                    dst.at[pl.ds(hbase + s * FLUSH, FLUSH)])
    plsc.subcore_barrier()


_dist_a = pl.kernel(
    _dist_a_body,
    out_type=(jax.ShapeDtypeStruct((N,), jnp.int32),
              jax.ShapeDtypeStruct((N,), jnp.int32)),
    mesh=_mesh,
    compiler_params=_cp,
    scratch_types=[
        pltpu.VMEM((W,), jnp.int32),
        pltpu.VMEM((W,), jnp.int32),
        pltpu.VMEM((W,), jnp.int32),
        pltpu.VMEM((W,), jnp.int32),
        pltpu.VMEM((W,), jnp.int32),
        pltpu.VMEM((W,), jnp.int32),
        pltpu.VMEM((W,), jnp.int32),
        pltpu.VMEM_SHARED((H + 256,), jnp.int32),
        pltpu.SemaphoreType.DMA,
    ],
)


def _dist_b_body(rcol, ccol, pos, eir, eic,
                 rv2, cv2, posv, rbuf, cbuf, s1buf, s2buf, spd, sem):
  c = lax.axis_index("c")
  s = lax.axis_index("s")
  ii = _iota16()

  hbase = c * H
  for rnd in range(2):
    for half in range(2):
      k = 2 * s + half

      @pl.loop(0, NW)
      def _win(win):
        base = k * CHUNK + win * W
        src_ref = rcol if rnd == 0 else ccol
        pltpu.sync_copy(src_ref.at[pl.ds(base, W)], rv2)
        pltpu.sync_copy(pos.at[pl.ds(base, W)], posv)

        @pl.loop(0, VPW, unroll=4)
        def _vreg(j):
          sl = pl.ds(pl.multiple_of(j * 16, 16), 16)
          p = posv[sl]
          loc = p - hbase
          owned = (loc >= 0) & (loc < H)
          s1buf[sl] = jnp.where(owned, loc, DUMP + ii)
          rbuf[sl] = rv2[sl]
        pltpu.async_copy(rbuf, spd.at[s1buf], sem).wait()

    plsc.subcore_barrier()
    dst = eir if rnd == 0 else eic
    pltpu.sync_copy(spd.at[pl.ds(s * FLUSH, FLUSH)],
                    dst.at[pl.ds(hbase + s * FLUSH, FLUSH)])
    plsc.subcore_barrier()


_dist_b = pl.kernel(
    _dist_b_body,
    out_type=(jax.ShapeDtypeStruct((N,), jnp.int32),
              jax.ShapeDtypeStruct((N,), jnp.int32)),
    mesh=_mesh,
    compiler_params=_cp,
    scratch_types=[
        pltpu.VMEM((W,), jnp.int32),
        pltpu.VMEM((W,), jnp.int32),
        pltpu.VMEM((W,), jnp.int32),
        pltpu.VMEM((W,), jnp.int32),
        pltpu.VMEM((W,), jnp.int32),
        pltpu.VMEM((W,), jnp.int32),
        pltpu.VMEM((W,), jnp.int32),
        pltpu.VMEM_SHARED((H + 256,), jnp.int32),
        pltpu.SemaphoreType.DMA,
    ],
)


@jax.jit
def kernel(indices, values):
  bcol = indices[:, 0]
  rcol = indices[:, 1]
  ccol = indices[:, 2]
  vals_i32 = lax.bitcast_convert_type(values, jnp.int32)
  zeros = jnp.zeros((T * B,), jnp.int32)

  ht1 = _hist_b(bcol)
  s1 = _scan1(ht1)
  key2a, idxa, ht2 = _pass1(bcol, rcol, s1, zeros)
  s2 = _scan2(ht2[: T * B], ht2[T * B:])
  key2b, idxb, ht3 = _pass2(key2a, idxa, s2, zeros)
  s3 = _scan2(ht3[: T * B], ht3[T * B:])
  pos = _pass3(key2b, idxb, s3)
  rid, ew_i32 = _dist_a(bcol, vals_i32, pos)
  eir, eic = _dist_b(rcol, ccol, pos)

  edge_index = jnp.stack([eir, eic], axis=1)
  edge_weight = lax.bitcast_convert_type(ew_i32, jnp.float32)[:, None]
  return edge_index, rid, edge_weight


# speedup vs baseline: 1.0706x; 1.0706x over previous
import functools
import jax
import jax.numpy as jnp
from jax import lax
from jax.experimental import pallas as pl
from jax.experimental.pallas import tpu as pltpu, tpu_sc as plsc

N = 2097152
B = 2048
NC = 2
NS = 16
T = NC * NS
CHUNK = N // T
W = 4096
NW = CHUNK // W
VPW = W // 16
SCAN_BINS = B // NS
H = N // 2
FLUSH = H // NS

_mesh = plsc.VectorSubcoreMesh(core_axis_name="c", subcore_axis_name="s")
_cp = pltpu.CompilerParams(needs_layout_passes=False, use_tc_tiling_on_sc=True)


def _iota16():
  return lax.iota(jnp.int32, 16)


def _col16(v):
  return jnp.full((16,), v, jnp.int32)


def _fill(ref, n, value):
  v = jnp.full((16,), value, jnp.int32)
  for i in range(n // 16):
    ref[pl.ds(i * 16, 16)] = v


def _build_idx(ref, n, scale, offset):
  ii = _iota16()
  for i in range(n // 16):
    ref[pl.ds(i * 16, 16)] = (ii + i * 16) * scale + offset


def _hist_b_body(bcol, ht, bufv, histv):
  w = lax.axis_index("c") * NS + lax.axis_index("s")
  _fill(histv, B, 0)

  @pl.loop(0, NW)
  def _win(win):
    pltpu.sync_copy(bcol.at[pl.ds(w * CHUNK + win * W, W)], bufv)
    for j in range(VPW):
      b16 = bufv[pl.ds(j * 16, 16)]
      cnt, lastm = plsc.scan_count(b16)
      plsc.addupdate_scatter(histv, [b16], cnt, mask=lastm)

  pltpu.sync_copy(histv, ht.at[pl.ds(w * B, B)])


_hist_b = pl.kernel(
    _hist_b_body,
    out_type=jax.ShapeDtypeStruct((T * B,), jnp.int32),
    mesh=_mesh,
    compiler_params=_cp,
    scratch_types=[
        pltpu.VMEM((W,), jnp.int32),
        pltpu.VMEM((B,), jnp.int32),
    ],
)


def _scan_body(two_inputs, *refs):
  if two_inputs:
    h0, h1, s_out, h0v, h1v, sv, totv, shv, sh = refs
  else:
    h0, s_out, h0v, sv, totv, shv, sh = refs
  c = lax.axis_index("c")
  s = lax.axis_index("s")
  ii = _iota16()

  @pl.when(c == 0)
  def _():
    for t in range(T):
      src = pl.ds(t * B + s * SCAN_BINS, SCAN_BINS)
      dst = pl.ds(t * SCAN_BINS, SCAN_BINS)
      pltpu.sync_copy(h0.at[src], h0v.at[dst])
      if two_inputs:
        pltpu.sync_copy(h1.at[src], h1v.at[dst])
    if two_inputs:
      for i in range(T * SCAN_BINS // 16):
        sl = pl.ds(i * 16, 16)
        h0v[sl] = h0v[sl] + h1v[sl]

    @pl.loop(0, SCAN_BINS, init_carry=jnp.int32(0))
    def _bin(bin_local, carry):
      lo = plsc.load_gather(h0v, [ii * SCAN_BINS + bin_local])
      hi = plsc.load_gather(h0v, [(ii + 16) * SCAN_BINS + bin_local])
      cs_lo = plsc.cumsum(lo)
      cs_hi = plsc.cumsum(hi)
      tot_lo = jnp.sum(lo)
      off = pl.multiple_of(bin_local * T, T)
      sv[pl.ds(off, 16)] = cs_lo - lo + carry
      sv[pl.ds(off + 16, 16)] = cs_hi - hi + (carry + tot_lo)
      return carry + tot_lo + jnp.sum(hi)

    total = _bin
    totv[pl.ds(0, 16)] = jnp.where(ii == 0, total, 0)
    pltpu.sync_copy(totv.at[pl.ds(0, 8)], sh.at[pl.ds(s * 8, 8)])

  plsc.subcore_barrier()

  @pl.when(c == 0)
  def _():
    pltpu.sync_copy(sh, shv)
    totals = plsc.load_gather(shv, [ii * 8])
    base = jnp.sum(jnp.where(ii < s, totals, 0))
    for i in range(T * SCAN_BINS // 16):
      sl = pl.ds(i * 16, 16)
      sv[sl] = sv[sl] + base
    pltpu.sync_copy(sv, s_out.at[pl.ds(s * T * SCAN_BINS, T * SCAN_BINS)])


def _make_scan(two_inputs):
  scratch = [
      pltpu.VMEM((T * SCAN_BINS,), jnp.int32),
      pltpu.VMEM((T * SCAN_BINS,), jnp.int32),
      pltpu.VMEM((16,), jnp.int32),
      pltpu.VMEM((NS * 8,), jnp.int32),
      pltpu.VMEM_SHARED((NS * 8,), jnp.int32),
  ]
  if two_inputs:
    scratch.insert(1, pltpu.VMEM((T * SCAN_BINS,), jnp.int32))
  return pl.kernel(
      functools.partial(_scan_body, two_inputs),
      out_type=jax.ShapeDtypeStruct((T * B,), jnp.int32),
      mesh=_mesh,
      compiler_params=_cp,
      scratch_types=scratch,
  )


_scan1 = _make_scan(False)
_scan2 = _make_scan(True)


DUMP = H
HIST0 = H + 256
SP_WORDS = HIST0 + T * B


def _pass_body(pass_id, *refs):
  if pass_id == 1:
    (bcol, rcol, s_in, zeros, out_a, out_b, ht_next,
     bcv, rcv, splitsv, offv, gidxv, abuf, bbuf, s1buf, s2buf, hbuf, onesbuf,
     spd, sem) = refs
  elif pass_id == 2:
    (in_a, in_b, s_in, zeros, out_a, out_b, ht_next,
     av, bv, offv, gidxv, abuf, bbuf, s1buf, s2buf, hbuf, onesbuf,
     spd, sem) = refs
  else:
    (in_a, in_b, s_in, pos_out,
     av, bv, offv, gidxv, abuf, s1buf,
     spd, sem) = refs
  c = lax.axis_index("c")
  s = lax.axis_index("s")
  ii = _iota16()

  if pass_id != 3:
    @pl.when(s == 0)
    def _():
      pltpu.sync_copy(zeros, spd.at[pl.ds(HIST0, T * B)])
    if pass_id == 1:
      _build_idx(gidxv, B, T, 0)
      pltpu.async_copy(s_in.at[gidxv], splitsv, sem).wait()
    _fill(onesbuf, W, 1)
  plsc.subcore_barrier()

  hbase = c * H
  n_rounds = 1 if pass_id == 3 else 2
  for rnd in range(n_rounds):
    for half in range(2):
      k = 2 * s + half
      _build_idx(gidxv, B, T, k)
      pltpu.async_copy(s_in.at[gidxv], offv, sem).wait()

      @pl.loop(0, NW)
      def _win(win):
        base = k * CHUNK + win * W
        if pass_id == 1:
          pltpu.sync_copy(bcol.at[pl.ds(base, W)], bcv)
          if rnd == 0:
            pltpu.sync_copy(rcol.at[pl.ds(base, W)], rcv)
        else:
          pltpu.sync_copy(in_a.at[pl.ds(base, W)], av)
          if pass_id == 3 or rnd == 1:
            pltpu.sync_copy(in_b.at[pl.ds(base, W)], bv)

        @pl.loop(0, VPW, unroll=4)
        def _vreg(j):
          sl = pl.ds(pl.multiple_of(j * 16, 16), 16)
          if pass_id == 1:
            b16 = bcv[sl]
            d16 = b16
            if rnd == 0:
              key2 = rcv[sl] + plsc.load_gather(splitsv, [b16])
            idxval = base + j * 16 + ii
          else:
            key2 = av[sl]
            idxval = bv[sl]
            d16 = (key2 & (B - 1)) if pass_id == 2 else (key2 >> 11)
          cnt, lastm = plsc.scan_count(d16)
          dest = plsc.load_gather(offv, [d16]) + (cnt - 1)
          plsc.addupdate_scatter(offv, [d16], cnt, mask=lastm)
          if pass_id == 3:
            loc = idxval - hbase
            owned = (loc >= 0) & (loc < H)
            s1buf[sl] = jnp.where(owned, loc, DUMP + ii)
            abuf[sl] = dest
          else:
            loc = dest - hbase
            owned = (loc >= 0) & (loc < H)
            s1buf[sl] = jnp.where(owned, loc, DUMP + ii)
            abuf[sl] = key2 if rnd == 0 else idxval
            if rnd == 0:
              nxt = (key2 & (B - 1)) if pass_id == 1 else (key2 >> 11)
              hbuf[sl] = HIST0 + ((dest >> 16) << 11) + nxt
        d1 = pltpu.async_copy(abuf, spd.at[s1buf], sem)
        if pass_id != 3 and rnd == 0:
          @pl.when(c == 0)
          def _():
            pltpu.sync_copy(onesbuf, spd.at[hbuf], add=True)
        d1.wait()

    plsc.subcore_barrier()
    if pass_id == 3:
      dst = pos_out
    else:
      dst = out_a if rnd == 0 else out_b
    pltpu.sync_copy(spd.at[pl.ds(s * FLUSH, FLUSH)],
                    dst.at[pl.ds(hbase + s * FLUSH, FLUSH)])
    plsc.subcore_barrier()

  if pass_id != 3:
    @pl.when(s == 0)
    def _():
      pltpu.sync_copy(spd.at[pl.ds(HIST0, T * B)],
                      ht_next.at[pl.ds(c * (T * B), T * B)])


def _make_pass(pass_id):
  plane = jax.ShapeDtypeStruct((N,), jnp.int32)
  if pass_id == 3:
    out_type = plane
    scratch = [
        pltpu.VMEM((W,), jnp.int32),
        pltpu.VMEM((W,), jnp.int32),
        pltpu.VMEM((B,), jnp.int32),
        pltpu.VMEM((B,), jnp.int32),
        pltpu.VMEM((W,), jnp.int32),
        pltpu.VMEM((W,), jnp.int32),
        pltpu.VMEM_SHARED((H + 256,), jnp.int32),
        pltpu.SemaphoreType.DMA,
    ]
  else:
    out_type = (plane, plane,
                jax.ShapeDtypeStruct((NC * T * B,), jnp.int32))
    first = ([pltpu.VMEM((W,), jnp.int32), pltpu.VMEM((W,), jnp.int32),
              pltpu.VMEM((B,), jnp.int32)]
             if pass_id == 1 else
             [pltpu.VMEM((W,), jnp.int32), pltpu.VMEM((W,), jnp.int32)])
    scratch = first + [
        pltpu.VMEM((B,), jnp.int32),
        pltpu.VMEM((B,), jnp.int32),
        pltpu.VMEM((W,), jnp.int32),
        pltpu.VMEM((W,), jnp.int32),
        pltpu.VMEM((W,), jnp.int32),
        pltpu.VMEM((W,), jnp.int32),
        pltpu.VMEM((W,), jnp.int32),
        pltpu.VMEM((W,), jnp.int32),
        pltpu.VMEM_SHARED((SP_WORDS,), jnp.int32),
        pltpu.SemaphoreType.DMA,
    ]
  return pl.kernel(
      functools.partial(_pass_body, pass_id),
      out_type=out_type,
      mesh=_mesh,
      compiler_params=_cp,
      scratch_types=scratch,
  )


_pass1 = _make_pass(1)
_pass2 = _make_pass(2)
_pass3 = _make_pass(3)


def _dist_a_body(bcol, vals, pos, rid, ew,
                 bufv, vv, posv, bbuf, vbuf, s1buf, s2buf, spd, sem):
  c = lax.axis_index("c")
  s = lax.axis_index("s")
  ii = _iota16()

  hbase = c * H
  for rnd in range(2):
    for half in range(2):
      k = 2 * s + half

      @pl.loop(0, NW)
      def _win(win):
        base = k * CHUNK + win * W
        src_ref = bcol if rnd == 0 else vals
        pltpu.sync_copy(src_ref.at[pl.ds(base, W)], vv)
        pltpu.sync_copy(pos.at[pl.ds(base, W)], posv)

        @pl.loop(0, VPW, unroll=4)
        def _vreg(j):
          sl = pl.ds(pl.multiple_of(j * 16, 16), 16)
          p = posv[sl]
          loc = p - hbase
          owned = (loc >= 0) & (loc < H)
          s1buf[sl] = jnp.where(owned, loc, DUMP + ii)
          vbuf[sl] = vv[sl]
        pltpu.async_copy(vbuf, spd.at[s1buf], sem).wait()

    plsc.subcore_barrier()
    dst = rid if rnd == 0 else ew
    pltpu.sync_copy(spd.at[pl.ds(s * FLUSH, FLUSH)],
                    dst.at[pl.ds(hbase + s * FLUSH, FLUSH)])
    plsc.subcore_barrier()


_dist_a = pl.kernel(
    _dist_a_body,
    out_type=(jax.ShapeDtypeStruct((N,), jnp.int32),
              jax.ShapeDtypeStruct((N,), jnp.int32)),
    mesh=_mesh,
    compiler_params=_cp,
    scratch_types=[
        pltpu.VMEM((W,), jnp.int32),
        pltpu.VMEM((W,), jnp.int32),
        pltpu.VMEM((W,), jnp.int32),
        pltpu.VMEM((W,), jnp.int32),
        pltpu.VMEM((W,), jnp.int32),
        pltpu.VMEM((W,), jnp.int32),
        pltpu.VMEM((W,), jnp.int32),
        pltpu.VMEM_SHARED((H + 256,), jnp.int32),
        pltpu.SemaphoreType.DMA,
    ],
)


def _dist_b_body(rcol, ccol, pos, eir, eic,
                 rv2, cv2, posv, rbuf, cbuf, s1buf, s2buf, spd, sem):
  c = lax.axis_index("c")
  s = lax.axis_index("s")
  ii = _iota16()

  hbase = c * H
  for rnd in range(2):
    for half in range(2):
      k = 2 * s + half

      @pl.loop(0, NW)
      def _win(win):
        base = k * CHUNK + win * W
        src_ref = rcol if rnd == 0 else ccol
        pltpu.sync_copy(src_ref.at[pl.ds(base, W)], rv2)
        pltpu.sync_copy(pos.at[pl.ds(base, W)], posv)

        @pl.loop(0, VPW, unroll=4)
        def _vreg(j):
          sl = pl.ds(pl.multiple_of(j * 16, 16), 16)
          p = posv[sl]
          loc = p - hbase
          owned = (loc >= 0) & (loc < H)
          s1buf[sl] = jnp.where(owned, loc, DUMP + ii)
          rbuf[sl] = rv2[sl]
        pltpu.async_copy(rbuf, spd.at[s1buf], sem).wait()

    plsc.subcore_barrier()
    dst = eir if rnd == 0 else eic
    pltpu.sync_copy(spd.at[pl.ds(s * FLUSH, FLUSH)],
                    dst.at[pl.ds(hbase + s * FLUSH, FLUSH)])
    plsc.subcore_barrier()


_dist_b = pl.kernel(
    _dist_b_body,
    out_type=(jax.ShapeDtypeStruct((N,), jnp.int32),
              jax.ShapeDtypeStruct((N,), jnp.int32)),
    mesh=_mesh,
    compiler_params=_cp,
    scratch_types=[
        pltpu.VMEM((W,), jnp.int32),
        pltpu.VMEM((W,), jnp.int32),
        pltpu.VMEM((W,), jnp.int32),
        pltpu.VMEM((W,), jnp.int32),
        pltpu.VMEM((W,), jnp.int32),
        pltpu.VMEM((W,), jnp.int32),
        pltpu.VMEM((W,), jnp.int32),
        pltpu.VMEM_SHARED((H + 256,), jnp.int32),
        pltpu.SemaphoreType.DMA,
    ],
)


@jax.jit
def kernel(indices, values):
  bcol = indices[:, 0]
  rcol = indices[:, 1]
  ccol = indices[:, 2]
  vals_i32 = lax.bitcast_convert_type(values, jnp.int32)
  zeros = jnp.zeros((T * B,), jnp.int32)

  ht1 = _hist_b(bcol)
  s1 = _scan1(ht1)
  key2a, idxa, ht2 = _pass1(bcol, rcol, s1, zeros)
  s2 = _scan2(ht2[: T * B], ht2[T * B:])
  key2b, idxb, ht3 = _pass2(key2a, idxa, s2, zeros)
  s3 = _scan2(ht3[: T * B], ht3[T * B:])
  pos = _pass3(key2b, idxb, s3)
  rid, ew_i32 = _dist_a(bcol, vals_i32, pos)
  eir, eic = _dist_b(rcol, ccol, pos)

  edge_index = jnp.stack([eir, eic], axis=1)
  edge_weight = lax.bitcast_convert_type(ew_i32, jnp.float32)[:, None]
  return edge_index, rid, edge_weight
